# baseline (speedup 1.0000x reference)
import jax
import jax.numpy as jnp
from jax import lax
from jax.experimental import pallas as pl
from jax.experimental.pallas import tpu as pltpu
from jax.experimental.pallas import tpu_sc as plsc

_FACTORS = 64
_LANES = 16
_NUM_WORKERS = 32
_BATCH = 16384
_BPW = _BATCH // _NUM_WORKERS
_HALF = _BPW // 2
_ROWS = _BPW // 128


def _bias_body(users_hbm, items_hbm, lb_hbm, rb_hbm, part_hbm,
               uidx_v, iidx_v, uhi_v, ihi_v, ub_v, rb_v, out_v, sem):
    wid = lax.axis_index("s") * 2 + lax.axis_index("c")
    base = wid * _BPW
    a0 = wid * _ROWS
    pltpu.sync_copy(users_hbm.at[pl.ds(a0, _ROWS)], uidx_v)
    pltpu.sync_copy(items_hbm.at[pl.ds(a0, _ROWS)], iidx_v)

    def shift(g, carry):
        uvec = uidx_v[g // 8, pl.ds((g % 8) * _LANES, _LANES)]
        ivec = iidx_v[g // 8, pl.ds((g % 8) * _LANES, _LANES)]
        uhi_v[pl.ds(g * _LANES, _LANES)] = uvec >> 4
        ihi_v[pl.ds(g * _LANES, _LANES)] = ivec >> 4
        return carry

    lax.fori_loop(0, _BPW // _LANES, shift, 0)

    c2 = pltpu.async_copy(lb_hbm.at[uhi_v], ub_v, sem)
    c3 = pltpu.async_copy(rb_hbm.at[ihi_v], rb_v, sem)
    c2.wait()
    c3.wait()

    def group(g, carry):
        rows = g * _LANES + lax.iota(jnp.int32, _LANES)
        uvec = uidx_v[g // 8, pl.ds((g % 8) * _LANES, _LANES)]
        ivec = iidx_v[g // 8, pl.ds((g % 8) * _LANES, _LANES)]
        ub = plsc.load_gather(ub_v, [rows, uvec & 15])
        rb = plsc.load_gather(rb_v, [rows, ivec & 15])
        out_v[pl.ds(g * _LANES, _LANES)] = ub + rb
        return carry

    lax.fori_loop(0, _BPW // _LANES, group, 0)
    pltpu.sync_copy(out_v, part_hbm.at[pl.ds(base, _BPW)])


def _dot_body(users_hbm, items_hbm, l_hbm, r_hbm, part_hbm, out_hbm,
              uidx_v, iidx_v, part_v, urows_v, irows_v, out_v, sem):
    wid = lax.axis_index("s") * 2 + lax.axis_index("c")
    base = wid * _BPW
    a0 = wid * _ROWS
    pltpu.sync_copy(users_hbm.at[pl.ds(a0, _ROWS)], uidx_v)
    pltpu.sync_copy(items_hbm.at[pl.ds(a0, _ROWS)], iidx_v)
    pltpu.sync_copy(part_hbm.at[pl.ds(base, _BPW)], part_v)

    ngrp = _HALF // _LANES
    for h in range(2):
        hb = h * _HALF

        def fire(g, carry):
            gg = (hb // _LANES) + g
            uvec = uidx_v[gg // 8, pl.ds((gg % 8) * _LANES, _LANES)]
            ivec = iidx_v[gg // 8, pl.ds((gg % 8) * _LANES, _LANES)]
            for j in range(_LANES):
                p = g * _LANES + j
                pltpu.make_async_copy(
                    l_hbm.at[uvec[j] >> 3, pl.ds(uvec[j] & 7, 1)],
                    urows_v.at[pl.ds(p, 1)], sem).start()
                pltpu.make_async_copy(
                    r_hbm.at[pl.ds(ivec[j], 1)],
                    irows_v.at[pl.ds(p, 1)], sem).start()
            return carry

        lax.fori_loop(0, ngrp, fire, 0)

        def drain(p, carry):
            pltpu.make_async_copy(
                l_hbm.at[0, pl.ds(0, 1)], urows_v.at[pl.ds(p, 1)],
                sem).wait()
            pltpu.make_async_copy(
                r_hbm.at[pl.ds(0, 1)], irows_v.at[pl.ds(p, 1)],
                sem).wait()
            return carry

        lax.fori_loop(0, _HALF, drain, 0)

        def group(g, carry):
            rows = g * _LANES + lax.iota(jnp.int32, _LANES)
            accs = [jnp.zeros((_LANES,), jnp.float32) for _ in range(4)]
            for k in range(_FACTORS):
                col = jnp.full((_LANES,), k, jnp.int32)
                uk = plsc.load_gather(urows_v, [rows, col])
                ik = plsc.load_gather(irows_v, [rows, col])
                accs[k % 4] = accs[k % 4] + uk * ik
            dot = (accs[0] + accs[1]) + (accs[2] + accs[3])
            pslice = part_v[pl.ds(hb + g * _LANES, _LANES)]
            out_v[pl.ds(hb + g * _LANES, _LANES)] = dot + pslice
            return carry

        lax.fori_loop(0, ngrp, group, 0)

    pltpu.sync_copy(out_v, out_hbm.at[pl.ds(base, _BPW)])


def kernel(minibatch, L, R, L_bias, R_bias):
    cols = minibatch.T.reshape(2, _BATCH // 128, 128)
    users3 = cols[0]
    items3 = cols[1]
    lb16 = L_bias.reshape(L_bias.shape[0] // _LANES, _LANES)
    rb16 = R_bias.reshape(R_bias.shape[0] // _LANES, _LANES)
    mesh = plsc.VectorSubcoreMesh(core_axis_name="c", subcore_axis_name="s")

    bias_f = pl.kernel(
        _bias_body,
        out_type=jax.ShapeDtypeStruct((_BATCH,), jnp.float32),
        mesh=mesh,
        scratch_types=[
            pltpu.VMEM((_ROWS, 128), jnp.int32),
            pltpu.VMEM((_ROWS, 128), jnp.int32),
            pltpu.VMEM((_BPW,), jnp.int32),
            pltpu.VMEM((_BPW,), jnp.int32),
            pltpu.VMEM((_BPW, _LANES), jnp.float32),
            pltpu.VMEM((_BPW, _LANES), jnp.float32),
            pltpu.VMEM((_BPW,), jnp.float32),
            pltpu.SemaphoreType.DMA,
        ],
        compiler_params=pltpu.CompilerParams(
            needs_layout_passes=False, use_tc_tiling_on_sc=False
        ),
    )
    partial = bias_f(users3, items3, lb16, rb16)

    l3 = L.reshape(L.shape[0] // 8, 8, _FACTORS)
    dot_f = pl.kernel(
        _dot_body,
        out_type=jax.ShapeDtypeStruct((_BATCH,), jnp.float32),
        mesh=mesh,
        scratch_types=[
            pltpu.VMEM((_ROWS, 128), jnp.int32),
            pltpu.VMEM((_ROWS, 128), jnp.int32),
            pltpu.VMEM((_BPW,), jnp.float32),
            pltpu.VMEM((_HALF, _FACTORS), jnp.float32),
            pltpu.VMEM((_HALF, _FACTORS), jnp.float32),
            pltpu.VMEM((_BPW,), jnp.float32),
            pltpu.SemaphoreType.DMA,
        ],
        compiler_params=pltpu.CompilerParams(
            needs_layout_passes=False, use_tc_tiling_on_sc=True
        ),
    )
    return dot_f(users3, items3, l3, R, partial)


# speedup vs baseline: 1.1309x; 1.1309x over previous
import jax
import jax.numpy as jnp
from jax import lax
from jax.experimental import pallas as pl
from jax.experimental.pallas import tpu as pltpu
from jax.experimental.pallas import tpu_sc as plsc

_FACTORS = 64
_LANES = 16
_NUM_WORKERS = 32
_BATCH = 16384
_BPW = _BATCH // _NUM_WORKERS
_HALF = _BPW // 2


def _bias_body(users_hbm, items_hbm, uhi_hbm, ihi_hbm, lb_hbm, rb_hbm,
               out_hbm, uidx_v, iidx_v, uhi_v, ihi_v, ub_v, rb_v, out_v,
               sem):
    wid = lax.axis_index("s") * 2 + lax.axis_index("c")
    base = wid * _BPW
    pltpu.sync_copy(users_hbm.at[wid], uidx_v)
    pltpu.sync_copy(items_hbm.at[wid], iidx_v)
    pltpu.sync_copy(uhi_hbm.at[wid], uhi_v)
    pltpu.sync_copy(ihi_hbm.at[wid], ihi_v)
    c2 = pltpu.async_copy(lb_hbm.at[uhi_v], ub_v, sem)
    c3 = pltpu.async_copy(rb_hbm.at[ihi_v], rb_v, sem)
    c2.wait()
    c3.wait()

    def group(g, carry):
        rows = g * _LANES + lax.iota(jnp.int32, _LANES)
        ulo = uidx_v[pl.ds(g * _LANES, _LANES)] & 15
        ilo = iidx_v[pl.ds(g * _LANES, _LANES)] & 15
        ub = plsc.load_gather(ub_v, [rows, ulo])
        rb = plsc.load_gather(rb_v, [rows, ilo])
        out_v[pl.ds(g * _LANES, _LANES)] = ub + rb
        return carry

    lax.fori_loop(0, _BPW // _LANES, group, 0)
    pltpu.sync_copy(out_v, out_hbm.at[pl.ds(base, _BPW)])


def _dot_body(users_hbm, items_hbm, l_hbm, r_hbm, part_hbm, out_hbm,
              uidx_v, iidx_v, part_v, urows_v, irows_v, out_v, sem):
    wid = lax.axis_index("s") * 2 + lax.axis_index("c")
    base = wid * _BPW
    pltpu.sync_copy(users_hbm.at[wid], uidx_v)
    pltpu.sync_copy(items_hbm.at[wid], iidx_v)
    pltpu.sync_copy(part_hbm.at[pl.ds(base, _BPW)], part_v)

    ngrp = _HALF // _LANES
    for h in range(2):
        hb = h * _HALF

        def fire(g, carry):
            uvec = uidx_v[pl.ds(hb + g * _LANES, _LANES)]
            ivec = iidx_v[pl.ds(hb + g * _LANES, _LANES)]
            for j in range(_LANES):
                p = g * _LANES + j
                pltpu.make_async_copy(
                    l_hbm.at[uvec[j] >> 3, pl.ds(uvec[j] & 7, 1)],
                    urows_v.at[pl.ds(p, 1)], sem).start()
                pltpu.make_async_copy(
                    r_hbm.at[ivec[j] >> 3, pl.ds(ivec[j] & 7, 1)],
                    irows_v.at[pl.ds(p, 1)], sem).start()
            return carry

        lax.fori_loop(0, ngrp, fire, 0)

        def drain(p, carry):
            pltpu.make_async_copy(
                l_hbm.at[0, pl.ds(0, 1)], urows_v.at[pl.ds(p, 1)],
                sem).wait()
            pltpu.make_async_copy(
                r_hbm.at[0, pl.ds(0, 1)], irows_v.at[pl.ds(p, 1)],
                sem).wait()
            return carry

        lax.fori_loop(0, _HALF, drain, 0)

        def group(g, carry):
            rows = g * _LANES + lax.iota(jnp.int32, _LANES)
            accs = [jnp.zeros((_LANES,), jnp.float32) for _ in range(4)]
            for k in range(_FACTORS):
                col = jnp.full((_LANES,), k, jnp.int32)
                uk = plsc.load_gather(urows_v, [rows, col])
                ik = plsc.load_gather(irows_v, [rows, col])
                accs[k % 4] = accs[k % 4] + uk * ik
            dot = (accs[0] + accs[1]) + (accs[2] + accs[3])
            pslice = part_v[pl.ds(hb + g * _LANES, _LANES)]
            out_v[pl.ds(hb + g * _LANES, _LANES)] = dot + pslice
            return carry

        lax.fori_loop(0, ngrp, group, 0)
    pltpu.sync_copy(out_v, out_hbm.at[pl.ds(base, _BPW)])


def kernel(minibatch, L, R, L_bias, R_bias):
    users = minibatch[:, 0].reshape(_NUM_WORKERS, _BPW)
    items = minibatch[:, 1].reshape(_NUM_WORKERS, _BPW)
    lb16 = L_bias.reshape(L_bias.shape[0] // _LANES, _LANES)
    rb16 = R_bias.reshape(R_bias.shape[0] // _LANES, _LANES)
    mesh = plsc.VectorSubcoreMesh(core_axis_name="c", subcore_axis_name="s")

    bias_f = pl.kernel(
        _bias_body,
        out_type=jax.ShapeDtypeStruct((_BATCH,), jnp.float32),
        mesh=mesh,
        scratch_types=[
            pltpu.VMEM((_BPW,), jnp.int32),
            pltpu.VMEM((_BPW,), jnp.int32),
            pltpu.VMEM((_BPW,), jnp.int32),
            pltpu.VMEM((_BPW,), jnp.int32),
            pltpu.VMEM((_BPW, _LANES), jnp.float32),
            pltpu.VMEM((_BPW, _LANES), jnp.float32),
            pltpu.VMEM((_BPW,), jnp.float32),
            pltpu.SemaphoreType.DMA,
        ],
        compiler_params=pltpu.CompilerParams(
            needs_layout_passes=False, use_tc_tiling_on_sc=False
        ),
    )
    partial = bias_f(users, items, users >> 4, items >> 4, lb16, rb16)

    dot_f = pl.kernel(
        _dot_body,
        out_type=jax.ShapeDtypeStruct((_BATCH,), jnp.float32),
        mesh=mesh,
        scratch_types=[
            pltpu.VMEM((_BPW,), jnp.int32),
            pltpu.VMEM((_BPW,), jnp.int32),
            pltpu.VMEM((_BPW,), jnp.float32),
            pltpu.VMEM((_HALF, _FACTORS), jnp.float32),
            pltpu.VMEM((_HALF, _FACTORS), jnp.float32),
            pltpu.VMEM((_BPW,), jnp.float32),
            pltpu.SemaphoreType.DMA,
        ],
        compiler_params=pltpu.CompilerParams(
            needs_layout_passes=False, use_tc_tiling_on_sc=True
        ),
    )
    l3 = L.reshape(L.shape[0] // 8, 8, _FACTORS)
    r3 = R.reshape(R.shape[0] // 8, 8, _FACTORS)
    return dot_f(users, items, l3, r3, partial)
